# baseline (speedup 1.0000x reference)
import functools

import jax
import jax.numpy as jnp
from jax.experimental import pallas as pl
from jax.experimental.pallas import tpu as pltpu


_F32 = jnp.float32


def _encoder_kernel(adj_ref, x_ref, w1_ref, w2_ref,
                    zemb_ref, sq_ref, aux_ref,
                    adj_v, s2_v, w2p_v, sem, *, n, tk, mt, d_e, epsilon):
    k = pl.program_id(0)
    nk = pl.num_programs(0)
    th = tk // 2

    def blk_copy(i, h):
        return pltpu.make_async_copy(
            adj_ref.at[pl.ds(i * tk + h * th, th), :],
            adj_v.at[pl.ds(i * tk + h * th, th), :],
            sem.at[i, h])

    @pl.when(k == 0)
    def _():
        for i in range(nk):
            blk_copy(i, 0).start()
            blk_copy(i, 1).start()
        d_z = w2_ref.shape[1]
        w2p_v[...] = jnp.pad(w2_ref[...],
                             ((0, 0), (0, w2p_v.shape[1] - d_z)))

    blk_copy(k, 0).wait()
    blk_copy(k, 1).wait()

    rows = pl.ds(k * tk, tk)
    ab = adj_v[rows, :]
    t1 = jnp.dot(ab, x_ref[...], preferred_element_type=_F32)
    h = jnp.maximum(jnp.dot(t1, w1_ref[...], preferred_element_type=_F32),
                    0.0)
    s2_v[rows, :] = jnp.dot(h, w2p_v[...], preferred_element_type=_F32)

    @pl.when(k == nk - 1)
    def _():
        d_zp = s2_v.shape[1]
        lane = jax.lax.broadcasted_iota(jnp.int32, (1, d_zp), 1)
        lane_mask = (lane < d_e).astype(_F32)
        for m in range(n // mt):
            r2 = pl.ds(m * mt, mt)
            zm = jnp.dot(adj_v[r2, :], s2_v[...],
                         preferred_element_type=_F32)
            ze = zm * lane_mask
            zemb_ref[r2, :] = ze
            sq_t = jnp.sum(ze * ze, axis=1, keepdims=True)
            sq_ref[r2, :] = sq_t + epsilon
            aux_ref[0:1, r2] = sq_t.T
            aux_ref[1:2, r2] = zm[:, d_e:d_e + 1].T


def _encoder(x, adj, w1, w2, *, d_e, d_zp, epsilon, tk=1024, mt=512):
    n, d_in = x.shape
    d_h = w1.shape[1]
    d_z = w2.shape[1]
    grid = (n // tk,)
    return pl.pallas_call(
        functools.partial(_encoder_kernel, n=n, tk=tk, mt=mt, d_e=d_e,
                          epsilon=epsilon),
        out_shape=[
            jax.ShapeDtypeStruct((n, d_zp), _F32),
            jax.ShapeDtypeStruct((n, 1), _F32),
            jax.ShapeDtypeStruct((2, n), _F32),
        ],
        grid_spec=pltpu.PrefetchScalarGridSpec(
            num_scalar_prefetch=0,
            grid=grid,
            in_specs=[
                pl.BlockSpec(memory_space=pl.ANY),
                pl.BlockSpec((n, d_in), lambda k: (0, 0)),
                pl.BlockSpec((d_in, d_h), lambda k: (0, 0)),
                pl.BlockSpec((d_h, d_z), lambda k: (0, 0)),
            ],
            out_specs=[
                pl.BlockSpec((n, d_zp), lambda k: (0, 0)),
                pl.BlockSpec((n, 1), lambda k: (0, 0)),
                pl.BlockSpec((2, n), lambda k: (0, 0)),
            ],
            scratch_shapes=[
                pltpu.VMEM((n, n), _F32),
                pltpu.VMEM((n, d_zp), _F32),
                pltpu.VMEM((d_h, d_zp), _F32),
                pltpu.SemaphoreType.DMA((n // tk, 2)),
            ],
        ),
        compiler_params=pltpu.CompilerParams(
            dimension_semantics=("arbitrary",),
            vmem_limit_bytes=56 * 1024 * 1024,
        ),
    )(adj, x, w1, w2)


def _decoder_kernel(zemb_ref, sq_ref, aux_ref, o_ref, *, tm):
    i = pl.program_id(0)
    zr = zemb_ref[pl.ds(i * tm, tm), :] * -2.0
    x2m = jax.lax.dot_general(
        zr, zemb_ref[...], dimension_numbers=(((1,), (1,)), ((), ())),
        preferred_element_type=_F32)
    sqi = sq_ref[pl.ds(i * tm, tm), :]
    dist = sqi + aux_ref[0:1, :] + x2m
    o_ref[...] = aux_ref[1:2, :] - jnp.log(dist)


def _decoder(zemb, sq_col, aux, *, tm=512):
    n, d = zemb.shape
    grid = (n // tm,)
    return pl.pallas_call(
        functools.partial(_decoder_kernel, tm=tm),
        out_shape=jax.ShapeDtypeStruct((n, n), _F32),
        grid_spec=pltpu.PrefetchScalarGridSpec(
            num_scalar_prefetch=0,
            grid=grid,
            in_specs=[
                pl.BlockSpec((n, d), lambda i: (0, 0)),
                pl.BlockSpec((n, 1), lambda i: (0, 0)),
                pl.BlockSpec((2, n), lambda i: (0, 0)),
            ],
            out_specs=pl.BlockSpec((tm, n), lambda i: (i, 0)),
        ),
        compiler_params=pltpu.CompilerParams(
            dimension_semantics=("arbitrary",),
        ),
    )(zemb, sq_col, aux)


def kernel(x, adj, w1, w2):
    n, d_in = x.shape
    d_h = w1.shape[1]
    d_z = w2.shape[1]
    d_e = d_z - 1
    d_zp = 128

    f32 = _F32
    zemb, sq_col, aux = _encoder(x.astype(f32), adj.astype(f32),
                                 w1.astype(f32), w2.astype(f32),
                                 d_e=d_e, d_zp=d_zp, epsilon=0.01)
    return _decoder(zemb, sq_col, aux, tm=512)


# speedup vs baseline: 1.0036x; 1.0036x over previous
import functools

import jax
import jax.numpy as jnp
from jax.experimental import pallas as pl
from jax.experimental.pallas import tpu as pltpu


_F32 = jnp.float32


def _encoder_kernel(adj_ref, x_ref, w1_ref, w2_ref,
                    zemb_ref, sq_ref, aux_ref,
                    adj_v, s2_v, w2p_v, sem, *, n, tk, mt, d_e, epsilon):
    k = pl.program_id(0)
    nk = pl.num_programs(0)
    th = tk // 2

    def blk_copy(i, h):
        return pltpu.make_async_copy(
            adj_ref.at[pl.ds(i * tk + h * th, th), :],
            adj_v.at[pl.ds(i * tk + h * th, th), :],
            sem.at[i, h])

    @pl.when(k == 0)
    def _():
        for i in range(nk):
            blk_copy(i, 0).start()
            blk_copy(i, 1).start()
        d_z = w2_ref.shape[1]
        w2p_v[...] = jnp.pad(w2_ref[...],
                             ((0, 0), (0, w2p_v.shape[1] - d_z)))

    blk_copy(k, 0).wait()
    blk_copy(k, 1).wait()

    rows = pl.ds(k * tk, tk)
    ab = adj_v[rows, :]
    t1 = jnp.dot(ab, x_ref[...], preferred_element_type=_F32)
    h = jnp.maximum(jnp.dot(t1, w1_ref[...], preferred_element_type=_F32),
                    0.0)
    s2_v[rows, :] = jnp.dot(h, w2p_v[...], preferred_element_type=_F32)

    @pl.when(k == nk - 1)
    def _():
        d_zp = s2_v.shape[1]
        lane = jax.lax.broadcasted_iota(jnp.int32, (1, d_zp), 1)
        lane_mask = (lane < d_e).astype(_F32)
        for m in range(n // mt):
            r2 = pl.ds(m * mt, mt)
            zm = jnp.dot(adj_v[r2, :], s2_v[...],
                         preferred_element_type=_F32)
            ze = zm * lane_mask
            zemb_ref[r2, :] = ze
            sq_t = jnp.sum(ze * ze, axis=1, keepdims=True)
            sq_ref[r2, :] = sq_t + epsilon
            aux_ref[0:1, r2] = sq_t.T
            aux_ref[1:2, r2] = zm[:, d_e:d_e + 1].T


def _encoder(x, adj, w1, w2, *, d_e, d_zp, epsilon, tk=512, mt=512):
    n, d_in = x.shape
    d_h = w1.shape[1]
    d_z = w2.shape[1]
    grid = (n // tk,)
    return pl.pallas_call(
        functools.partial(_encoder_kernel, n=n, tk=tk, mt=mt, d_e=d_e,
                          epsilon=epsilon),
        out_shape=[
            jax.ShapeDtypeStruct((n, d_zp), _F32),
            jax.ShapeDtypeStruct((n, 1), _F32),
            jax.ShapeDtypeStruct((2, n), _F32),
        ],
        grid_spec=pltpu.PrefetchScalarGridSpec(
            num_scalar_prefetch=0,
            grid=grid,
            in_specs=[
                pl.BlockSpec(memory_space=pl.ANY),
                pl.BlockSpec((n, d_in), lambda k: (0, 0)),
                pl.BlockSpec((d_in, d_h), lambda k: (0, 0)),
                pl.BlockSpec((d_h, d_z), lambda k: (0, 0)),
            ],
            out_specs=[
                pl.BlockSpec((n, d_zp), lambda k: (0, 0)),
                pl.BlockSpec((n, 1), lambda k: (0, 0)),
                pl.BlockSpec((2, n), lambda k: (0, 0)),
            ],
            scratch_shapes=[
                pltpu.VMEM((n, n), _F32),
                pltpu.VMEM((n, d_zp), _F32),
                pltpu.VMEM((d_h, d_zp), _F32),
                pltpu.SemaphoreType.DMA((n // tk, 2)),
            ],
        ),
        compiler_params=pltpu.CompilerParams(
            dimension_semantics=("arbitrary",),
            vmem_limit_bytes=56 * 1024 * 1024,
        ),
    )(adj, x, w1, w2)


def _decoder_kernel(zemb_ref, sq_ref, aux_ref, o_ref, *, tm):
    i = pl.program_id(0)
    zr = zemb_ref[pl.ds(i * tm, tm), :] * -2.0
    x2m = jax.lax.dot_general(
        zr, zemb_ref[...], dimension_numbers=(((1,), (1,)), ((), ())),
        preferred_element_type=_F32)
    sqi = sq_ref[pl.ds(i * tm, tm), :]
    dist = sqi + aux_ref[0:1, :] + x2m
    o_ref[...] = aux_ref[1:2, :] - jnp.log(dist)


def _decoder(zemb, sq_col, aux, *, tm=512):
    n, d = zemb.shape
    grid = (n // tm,)
    return pl.pallas_call(
        functools.partial(_decoder_kernel, tm=tm),
        out_shape=jax.ShapeDtypeStruct((n, n), _F32),
        grid_spec=pltpu.PrefetchScalarGridSpec(
            num_scalar_prefetch=0,
            grid=grid,
            in_specs=[
                pl.BlockSpec((n, d), lambda i: (0, 0)),
                pl.BlockSpec((n, 1), lambda i: (0, 0)),
                pl.BlockSpec((2, n), lambda i: (0, 0)),
            ],
            out_specs=pl.BlockSpec((tm, n), lambda i: (i, 0)),
        ),
        compiler_params=pltpu.CompilerParams(
            dimension_semantics=("arbitrary",),
        ),
    )(zemb, sq_col, aux)


def kernel(x, adj, w1, w2):
    n, d_in = x.shape
    d_h = w1.shape[1]
    d_z = w2.shape[1]
    d_e = d_z - 1
    d_zp = 128

    f32 = _F32
    zemb, sq_col, aux = _encoder(x.astype(f32), adj.astype(f32),
                                 w1.astype(f32), w2.astype(f32),
                                 d_e=d_e, d_zp=d_zp, epsilon=0.01)
    return _decoder(zemb, sq_col, aux, tm=384)


# speedup vs baseline: 1.0923x; 1.0884x over previous
import functools

import jax
import jax.numpy as jnp
from jax.experimental import pallas as pl
from jax.experimental.pallas import tpu as pltpu


_F32 = jnp.float32


def _fused_kernel(adj_ref, x_ref, w1_ref, w2_ref, o_ref,
                  adj_v, s2_v, w2p_v, zemb_v, sq_v, aux_v, sem,
                  *, n, tk, mt, tm, d_e, epsilon):
    k = pl.program_id(0)
    nk = n // tk
    th = tk // 2

    def blk_copy(i, h):
        return pltpu.make_async_copy(
            adj_ref.at[pl.ds(i * tk + h * th, th), :],
            adj_v.at[pl.ds(i * tk + h * th, th), :],
            sem.at[i, h])

    @pl.when(k == 0)
    def _():
        for i in range(nk):
            blk_copy(i, 0).start()
            blk_copy(i, 1).start()
        d_z = w2_ref.shape[1]
        w2p_v[...] = jnp.pad(w2_ref[...],
                             ((0, 0), (0, w2p_v.shape[1] - d_z)))

    @pl.when(k < nk)
    def _():
        blk_copy(k, 0).wait()
        blk_copy(k, 1).wait()
        rows = pl.ds(k * tk, tk)
        ab = adj_v[rows, :]
        t1 = jnp.dot(ab, x_ref[...], preferred_element_type=_F32)
        h = jnp.maximum(
            jnp.dot(t1, w1_ref[...], preferred_element_type=_F32), 0.0)
        s2_v[rows, :] = jnp.dot(h, w2p_v[...], preferred_element_type=_F32)

    @pl.when(k == nk - 1)
    def _():
        d_zp = s2_v.shape[1]
        lane = jax.lax.broadcasted_iota(jnp.int32, (1, d_zp), 1)
        lane_mask = (lane < d_e).astype(_F32)
        for m in range(n // mt):
            r2 = pl.ds(m * mt, mt)
            zm = jnp.dot(adj_v[r2, :], s2_v[...],
                         preferred_element_type=_F32)
            ze = zm * lane_mask
            zemb_v[r2, :] = ze
            sq_t = jnp.sum(ze * ze, axis=1, keepdims=True)
            sq_v[r2, :] = sq_t + epsilon
            aux_v[0:1, r2] = sq_t.T
            aux_v[1:2, r2] = zm[:, d_e:d_e + 1].T

    @pl.when(k >= nk)
    def _():
        i = k - nk
        zr = zemb_v[pl.ds(i * tm, tm), :] * -2.0
        x2m = jax.lax.dot_general(
            zr, zemb_v[...], dimension_numbers=(((1,), (1,)), ((), ())),
            preferred_element_type=_F32)
        sqi = sq_v[pl.ds(i * tm, tm), :]
        dist = sqi + aux_v[0:1, :] + x2m
        o_ref[...] = aux_v[1:2, :] - jnp.log(dist)


def _gravity_gae(x, adj, w1, w2, *, d_e, d_zp, epsilon,
                 tk=512, mt=512, tm=384):
    n, d_in = x.shape
    d_h = w1.shape[1]
    d_z = w2.shape[1]
    nk = n // tk
    nd = n // tm
    grid = (nk + nd,)
    return pl.pallas_call(
        functools.partial(_fused_kernel, n=n, tk=tk, mt=mt, tm=tm,
                          d_e=d_e, epsilon=epsilon),
        out_shape=jax.ShapeDtypeStruct((n, n), _F32),
        grid_spec=pltpu.PrefetchScalarGridSpec(
            num_scalar_prefetch=0,
            grid=grid,
            in_specs=[
                pl.BlockSpec(memory_space=pl.ANY),
                pl.BlockSpec((n, d_in), lambda k: (0, 0)),
                pl.BlockSpec((d_in, d_h), lambda k: (0, 0)),
                pl.BlockSpec((d_h, d_z), lambda k: (0, 0)),
            ],
            out_specs=pl.BlockSpec(
                (tm, n), lambda k: (jnp.maximum(k - nk, 0), 0)),
            scratch_shapes=[
                pltpu.VMEM((n, n), _F32),
                pltpu.VMEM((n, d_zp), _F32),
                pltpu.VMEM((d_h, d_zp), _F32),
                pltpu.VMEM((n, d_zp), _F32),
                pltpu.VMEM((n, 1), _F32),
                pltpu.VMEM((2, n), _F32),
                pltpu.SemaphoreType.DMA((n // tk, 2)),
            ],
        ),
        compiler_params=pltpu.CompilerParams(
            dimension_semantics=("arbitrary",),
            vmem_limit_bytes=56 * 1024 * 1024,
        ),
    )(adj, x, w1, w2)


def kernel(x, adj, w1, w2):
    d_z = w2.shape[1]
    d_e = d_z - 1
    d_zp = 128

    f32 = _F32
    return _gravity_gae(x.astype(f32), adj.astype(f32),
                        w1.astype(f32), w2.astype(f32),
                        d_e=d_e, d_zp=d_zp, epsilon=0.01)


# speedup vs baseline: 1.1125x; 1.0184x over previous
import functools

import jax
import jax.numpy as jnp
from jax.experimental import pallas as pl
from jax.experimental.pallas import tpu as pltpu


_F32 = jnp.float32


def _fused_kernel(adj_ref, x_ref, w1_ref, w2_ref, o_ref,
                  adj_v, s2_v, w2p_v, zemb_v, sq_v, aux_v, sem,
                  *, n, tk, mt, tm, d_e, epsilon):
    k = pl.program_id(0)
    nk = n // tk
    th = tk // 2

    def blk_copy(i, h):
        return pltpu.make_async_copy(
            adj_ref.at[pl.ds(i * tk + h * th, th), :],
            adj_v.at[pl.ds(i * tk + h * th, th), :],
            sem.at[i, h])

    @pl.when(k == 0)
    def _():
        for i in range(nk):
            blk_copy(i, 0).start()
            blk_copy(i, 1).start()
        d_z = w2_ref.shape[1]
        w2p_v[...] = jnp.pad(w2_ref[...],
                             ((0, 0), (0, w2p_v.shape[1] - d_z)))

    @pl.when(k < nk)
    def _():
        blk_copy(k, 0).wait()
        blk_copy(k, 1).wait()
        rows = pl.ds(k * tk, tk)
        ab = adj_v[rows, :]
        t1 = jnp.dot(ab, x_ref[...], preferred_element_type=_F32)
        h = jnp.maximum(
            jnp.dot(t1, w1_ref[...], preferred_element_type=_F32), 0.0)
        s2_v[rows, :] = jnp.dot(h, w2p_v[...], preferred_element_type=_F32)

    @pl.when(k == nk - 1)
    def _():
        d_zp = s2_v.shape[1]
        lane = jax.lax.broadcasted_iota(jnp.int32, (1, d_zp), 1)
        lane_mask = (lane < d_e).astype(_F32)
        for m in range(n // mt):
            r2 = pl.ds(m * mt, mt)
            zm = jnp.dot(adj_v[r2, :], s2_v[...],
                         preferred_element_type=_F32)
            ze = zm * lane_mask
            zemb_v[r2, :] = ze
            sq_t = jnp.sum(ze * ze, axis=1, keepdims=True)
            sq_v[r2, :] = sq_t + epsilon
            aux_v[0:1, r2] = sq_t.T
            aux_v[1:2, r2] = zm[:, d_e:d_e + 1].T

    @pl.when(k >= nk)
    def _():
        i = k - nk
        zr = zemb_v[pl.ds(i * tm, tm), :] * -2.0
        x2m = jax.lax.dot_general(
            zr, zemb_v[...], dimension_numbers=(((1,), (1,)), ((), ())),
            preferred_element_type=_F32)
        sqi = sq_v[pl.ds(i * tm, tm), :]
        dist = sqi + aux_v[0:1, :] + x2m
        o_ref[...] = aux_v[1:2, :] - jnp.log(dist)


def _gravity_gae(x, adj, w1, w2, *, d_e, d_zp, epsilon,
                 tk=512, mt=512, tm=512):
    n, d_in = x.shape
    d_h = w1.shape[1]
    d_z = w2.shape[1]
    nk = n // tk
    nd = n // tm
    grid = (nk + nd,)
    return pl.pallas_call(
        functools.partial(_fused_kernel, n=n, tk=tk, mt=mt, tm=tm,
                          d_e=d_e, epsilon=epsilon),
        out_shape=jax.ShapeDtypeStruct((n, n), _F32),
        grid_spec=pltpu.PrefetchScalarGridSpec(
            num_scalar_prefetch=0,
            grid=grid,
            in_specs=[
                pl.BlockSpec(memory_space=pl.ANY),
                pl.BlockSpec((n, d_in), lambda k: (0, 0)),
                pl.BlockSpec((d_in, d_h), lambda k: (0, 0)),
                pl.BlockSpec((d_h, d_z), lambda k: (0, 0)),
            ],
            out_specs=pl.BlockSpec(
                (tm, n), lambda k: (jnp.maximum(k - nk, 0), 0)),
            scratch_shapes=[
                pltpu.VMEM((n, n), _F32),
                pltpu.VMEM((n, d_zp), _F32),
                pltpu.VMEM((d_h, d_zp), _F32),
                pltpu.VMEM((n, d_zp), _F32),
                pltpu.VMEM((n, 1), _F32),
                pltpu.VMEM((2, n), _F32),
                pltpu.SemaphoreType.DMA((n // tk, 2)),
            ],
        ),
        compiler_params=pltpu.CompilerParams(
            dimension_semantics=("arbitrary",),
            vmem_limit_bytes=56 * 1024 * 1024,
        ),
    )(adj, x, w1, w2)


def kernel(x, adj, w1, w2):
    d_z = w2.shape[1]
    d_e = d_z - 1
    d_zp = 128

    f32 = _F32
    return _gravity_gae(x.astype(f32), adj.astype(f32),
                        w1.astype(f32), w2.astype(f32),
                        d_e=d_e, d_zp=d_zp, epsilon=0.01)
